# baseline (speedup 1.0000x reference)
import jax
import jax.numpy as jnp
from jax import lax
from jax.experimental import pallas as pl
from jax.experimental.pallas import tpu as pltpu
from jax.experimental.pallas import tpu_sc as plsc

_NC, _NS, _L = 2, 16, 16
_NW = _NC * _NS
_B, _T, _C = 8, 512, 1024
_LOUT = 2048
_QPB = _NW // _B
_QT = _LOUT // _QPB
_R = 32
_NB = 3
_ZR = 16
_NCH = _QT // _R


def _body(x_hbm, dur_hbm, tl_hbm, out_hbm,
          tl_v, dur_v, hist_v, idx_v, buf_v, zbuf_v,
          gsem, ssem, msem):
    wid = lax.axis_index("s") * _NC + lax.axis_index("c")
    b = wid // _QPB
    q = wid - b * _QPB
    brow0 = b * _LOUT

    tl_cp = pltpu.make_async_copy(tl_hbm, tl_v, msem.at[0])
    tl_cp.start()
    dur_cp = pltpu.make_async_copy(dur_hbm.at[b], dur_v, msem.at[1])
    dur_cp.start()

    zeros_i = jnp.zeros((_L,), jnp.int32)
    ones_i = jnp.ones((_L,), jnp.int32)
    zeros_f = jnp.zeros((_L,), jnp.float32)

    def _zero_hist(i, c):
        for u in range(4):
            hist_v[pl.ds(i * 4 * _L + u * _L, _L)] = zeros_i
        return c
    lax.fori_loop(0, _LOUT // _L // 4, _zero_hist, 0)

    dur_cp.wait()
    tl_cp.wait()
    tl = jnp.max(tl_v[...])

    def _csum(i, carry):
        v = jnp.maximum(dur_v[pl.ds(i * _L, _L)], 1)
        s = plsc.cumsum(v) + carry
        plsc.addupdate_scatter(hist_v, [s], ones_i, mask=s < _LOUT)
        return jnp.max(s)
    total = lax.fori_loop(0, _T // _L, _csum, jnp.int32(0))
    limit = jnp.minimum(total, tl)

    base_row = b * _T
    def _psum(i, carry):
        ps = plsc.cumsum(hist_v[pl.ds(i * _L, _L)]) + carry
        idx_v[pl.ds(i * _L, _L)] = jnp.minimum(ps, _T - 1) + base_row
        return jnp.max(ps)
    lax.fori_loop(0, _LOUT // _L, _psum, jnp.int32(0))

    def _t0(c):
        return (q + c * _QPB) * _R

    def _gather(c, slot):
        return pltpu.make_async_copy(
            x_hbm.at[idx_v.at[pl.ds(_t0(c), _R)]],
            buf_v.at[slot],
            gsem.at[slot])

    def _out_slice(c):
        return out_hbm.at[pl.ds(brow0 + _t0(c), _R)]

    def _store_wait(c):
        pltpu.make_async_copy(buf_v.at[c % _NB], _out_slice(c), ssem.at[c % _NB]).wait()

    def _kk(c):
        return limit - _t0(c)

    @pl.when(_kk(0) > 0)
    def _():
        _gather(0, 0).start()

    @pl.when(_kk(_NCH - 1) <= 0)
    def _():
        def _zb(i, c):
            for u in range(_C // _L):
                zbuf_v[i, pl.ds(u * _L, _L)] = zeros_f
            return c
        lax.fori_loop(0, _ZR, _zb, 0)

    for c in range(_NCH):
        slot = c % _NB
        if c >= _NB - 1:
            _store_wait(c - (_NB - 1))
        if c + 1 < _NCH:
            @pl.when(_kk(c + 1) > 0)
            def _(c=c):
                _gather(c + 1, (c + 1) % _NB).start()
        kk = _kk(c)

        @pl.when(kk > 0)
        def _(c=c, slot=slot, kk=kk):
            _gather(c, slot).wait()

            @pl.when(kk < _R)
            def _():
                def _ztail(r, cc):
                    @pl.when(r >= kk)
                    def _():
                        def _zc(j, c2):
                            for u in range(4):
                                buf_v[slot, r, pl.ds(j * 4 * _L + u * _L, _L)] = zeros_f
                            return c2
                        lax.fori_loop(0, _C // _L // 4, _zc, 0)
                    return cc
                lax.fori_loop(0, _R, _ztail, 0)

            pltpu.make_async_copy(buf_v.at[slot], _out_slice(c), ssem.at[slot]).start()

        @pl.when(kk <= 0)
        def _(c=c, slot=slot):
            base = brow0 + _t0(c)
            pltpu.make_async_copy(
                zbuf_v, out_hbm.at[pl.ds(base, _ZR)], ssem.at[slot]).start()
            pltpu.make_async_copy(
                zbuf_v, out_hbm.at[pl.ds(base + _ZR, _ZR)], ssem.at[slot]).start()

    for c in range(_NCH - (_NB - 1), _NCH):
        _store_wait(c)


_sc_call = pl.kernel(
    _body,
    out_type=jax.ShapeDtypeStruct((_B * _LOUT, _C), jnp.float32),
    mesh=plsc.VectorSubcoreMesh(core_axis_name="c", subcore_axis_name="s",
                                num_cores=_NC, num_subcores=_NS),
    compiler_params=pltpu.CompilerParams(needs_layout_passes=False),
    scratch_types=[
        pltpu.VMEM((_L,), jnp.int32),
        pltpu.VMEM((_T,), jnp.int32),
        pltpu.VMEM((_LOUT,), jnp.int32),
        pltpu.VMEM((_LOUT,), jnp.int32),
        pltpu.VMEM((_NB, _R, _C), jnp.float32),
        pltpu.VMEM((_ZR, _C), jnp.float32),
        pltpu.SemaphoreType.DMA((_NB,)),
        pltpu.SemaphoreType.DMA((_NB,)),
        pltpu.SemaphoreType.DMA((2,)),
    ],
)


def kernel(x, durations, target_length):
    x2 = x.reshape(_B * _T, _C)
    dur = durations.astype(jnp.int32)
    tl = jnp.full((_L,), target_length, dtype=jnp.int32)
    out = _sc_call(x2, dur, tl)
    return out.reshape(_B, _LOUT, _C)


# speedup vs baseline: 1.0393x; 1.0393x over previous
import jax
import jax.numpy as jnp
from jax import lax
from jax.experimental import pallas as pl
from jax.experimental.pallas import tpu as pltpu
from jax.experimental.pallas import tpu_sc as plsc

_NC, _NS, _L = 2, 16, 16
_NW = _NC * _NS
_B, _T, _C = 8, 512, 1024
_LOUT = 2048
_QPB = _NW // _B
_QT = _LOUT // _QPB
_R = 32
_NB = 3
_ZR = 16
_NCH = _QT // _R


def _body(x_hbm, dur_hbm, tl_hbm, out_hbm,
          tl_v, dur_v, hist_v, idx_v, buf_v, zbuf_v,
          gsem, ssem, msem):
    wid = lax.axis_index("s") * _NC + lax.axis_index("c")
    b = wid // _QPB
    q = wid - b * _QPB
    brow0 = b * _LOUT

    tl_cp = pltpu.make_async_copy(tl_hbm, tl_v, msem.at[0])
    tl_cp.start()
    dur_cp = pltpu.make_async_copy(dur_hbm.at[b], dur_v, msem.at[1])
    dur_cp.start()

    zeros_i = jnp.zeros((_L,), jnp.int32)
    ones_i = jnp.ones((_L,), jnp.int32)
    zeros_f = jnp.zeros((_L,), jnp.float32)

    def _zero_hist(i, c):
        for u in range(4):
            hist_v[pl.ds(i * 4 * _L + u * _L, _L)] = zeros_i
        return c
    lax.fori_loop(0, _LOUT // _L // 4, _zero_hist, 0)

    dur_cp.wait()
    tl_cp.wait()
    tl = jnp.max(tl_v[...])

    def _csum(i, carry):
        v = jnp.maximum(dur_v[pl.ds(i * _L, _L)], 1)
        s = plsc.cumsum(v) + carry
        plsc.addupdate_scatter(hist_v, [s], ones_i, mask=s < _LOUT)
        return jnp.max(s)
    total = lax.fori_loop(0, _T // _L, _csum, jnp.int32(0))
    limit = jnp.minimum(total, tl)

    base_row = b * _T
    def _psum(i, carry):
        ps = plsc.cumsum(hist_v[pl.ds(i * _L, _L)]) + carry
        idx_v[pl.ds(i * _L, _L)] = jnp.minimum(ps, _T - 1) + base_row
        return jnp.max(ps)
    lax.fori_loop(0, _LOUT // _L, _psum, jnp.int32(0))

    def _t0(c):
        return (q + c * _QPB) * _R

    def _gather(c, slot):
        return pltpu.make_async_copy(
            x_hbm.at[idx_v.at[pl.ds(_t0(c), _R)]],
            buf_v.at[slot],
            gsem.at[slot])

    def _out_slice(c):
        return out_hbm.at[pl.ds(brow0 + _t0(c), _R)]

    def _store_wait(c):
        pltpu.make_async_copy(buf_v.at[c % _NB], _out_slice(c), ssem.at[c % _NB]).wait()

    def _kk(c):
        return limit - _t0(c)

    @pl.when(_kk(0) > 0)
    def _():
        _gather(0, 0).start()

    @pl.when(_kk(_NCH - 1) <= 0)
    def _():
        def _zb(i, c):
            for u in range(_C // _L):
                zbuf_v[i, pl.ds(u * _L, _L)] = zeros_f
            return c
        lax.fori_loop(0, _ZR, _zb, 0)

    def _chunk(c, carry):
        slot = c % _NB

        @pl.when(c >= _NB - 1)
        def _():
            _store_wait(c - (_NB - 1))

        @pl.when((c + 1 < _NCH) & (_kk(c + 1) > 0))
        def _():
            _gather(c + 1, (c + 1) % _NB).start()

        kk = _kk(c)

        @pl.when(kk > 0)
        def _():
            _gather(c, slot).wait()

            @pl.when(kk < _R)
            def _():
                def _ztail(r, cc):
                    @pl.when(r >= kk)
                    def _():
                        def _zc(j, c2):
                            for u in range(4):
                                buf_v[slot, r, pl.ds(j * 4 * _L + u * _L, _L)] = zeros_f
                            return c2
                        lax.fori_loop(0, _C // _L // 4, _zc, 0)
                    return cc
                lax.fori_loop(0, _R, _ztail, 0)

            pltpu.make_async_copy(buf_v.at[slot], _out_slice(c), ssem.at[slot]).start()

        @pl.when(kk <= 0)
        def _():
            base = brow0 + _t0(c)
            pltpu.make_async_copy(
                zbuf_v, out_hbm.at[pl.ds(base, _ZR)], ssem.at[slot]).start()
            pltpu.make_async_copy(
                zbuf_v, out_hbm.at[pl.ds(base + _ZR, _ZR)], ssem.at[slot]).start()

        return carry

    lax.fori_loop(0, _NCH, _chunk, 0)

    def _drain(c, carry):
        _store_wait(c)
        return carry
    lax.fori_loop(_NCH - (_NB - 1), _NCH, _drain, 0)


_sc_call = pl.kernel(
    _body,
    out_type=jax.ShapeDtypeStruct((_B * _LOUT, _C), jnp.float32),
    mesh=plsc.VectorSubcoreMesh(core_axis_name="c", subcore_axis_name="s",
                                num_cores=_NC, num_subcores=_NS),
    compiler_params=pltpu.CompilerParams(needs_layout_passes=False),
    scratch_types=[
        pltpu.VMEM((_L,), jnp.int32),
        pltpu.VMEM((_T,), jnp.int32),
        pltpu.VMEM((_LOUT,), jnp.int32),
        pltpu.VMEM((_LOUT,), jnp.int32),
        pltpu.VMEM((_NB, _R, _C), jnp.float32),
        pltpu.VMEM((_ZR, _C), jnp.float32),
        pltpu.SemaphoreType.DMA((_NB,)),
        pltpu.SemaphoreType.DMA((_NB,)),
        pltpu.SemaphoreType.DMA((2,)),
    ],
)


def kernel(x, durations, target_length):
    x2 = x.reshape(_B * _T, _C)
    dur = durations.astype(jnp.int32)
    tl = jnp.full((_L,), target_length, dtype=jnp.int32)
    out = _sc_call(x2, dur, tl)
    return out.reshape(_B, _LOUT, _C)
